# Initial kernel scaffold; baseline (speedup 1.0000x reference)
import functools

import jax
import jax.numpy as jnp
from jax import lax
from jax.experimental import pallas as pl
from jax.experimental.pallas import tpu as pltpu
from jax.experimental.pallas import tpu_sc as plsc

B = 16384
L = 200
EMBED = 16
NW = 32
ROWS_PER_W = B // NW
CH = 16
IDX_PER_CH = CH * L
GW = 128
G = IDX_PER_CH // GW
NCH = ROWS_PER_W // CH


def _make_kernel():
    mesh = plsc.VectorSubcoreMesh(core_axis_name="c", subcore_axis_name="s")

    @functools.partial(
        pl.kernel,
        mesh=mesh,
        out_type=jax.ShapeDtypeStruct((B, EMBED), jnp.float32),
        scratch_types=[
            pltpu.VMEM((G, GW), jnp.int32),
            pltpu.VMEM((IDX_PER_CH, EMBED), jnp.float32),
            pltpu.VMEM((CH, EMBED), jnp.float32),
            pltpu.SemaphoreType.DMA,
        ],
    )
    def k(x_hbm, table_hbm, out_hbm, idx_v, buf_v, out_v, sem):
        cid = lax.axis_index("c")
        sid = lax.axis_index("s")
        wid = sid * 2 + cid

        def chunk_body(c, carry):
            row0 = (wid * NCH + c) * G
            pltpu.sync_copy(x_hbm.at[pl.ds(row0, G)], idx_v)

            def fire(j, _):
                pltpu.make_async_copy(
                    table_hbm.at[idx_v.at[j]],
                    buf_v.at[pl.ds(j * GW, GW)],
                    sem,
                ).start()
                return _

            lax.fori_loop(0, G, fire, 0)

            def drain(j, _):
                pltpu.make_async_copy(
                    table_hbm.at[idx_v.at[j]],
                    buf_v.at[pl.ds(j * GW, GW)],
                    sem,
                ).wait()
                return _

            lax.fori_loop(0, G, drain, 0)

            def row_body(r, _):
                base = r * L

                def acc_body(i, accs):
                    a0, a1, a2, a3 = accs
                    o = base + i * 8
                    a0 = a0 + buf_v[o + 0]
                    a1 = a1 + buf_v[o + 1]
                    a2 = a2 + buf_v[o + 2]
                    a3 = a3 + buf_v[o + 3]
                    a0 = a0 + buf_v[o + 4]
                    a1 = a1 + buf_v[o + 5]
                    a2 = a2 + buf_v[o + 6]
                    a3 = a3 + buf_v[o + 7]
                    return (a0, a1, a2, a3)

                z = jnp.zeros((16,), jnp.float32)
                a0, a1, a2, a3 = lax.fori_loop(0, L // 8, acc_body, (z, z, z, z))
                out_v[r] = ((a0 + a1) + (a2 + a3)) * (1.0 / L)
                return _

            lax.fori_loop(0, CH, row_body, 0)

            out0 = wid * ROWS_PER_W + c * CH
            pltpu.sync_copy(out_v, out_hbm.at[pl.ds(out0, CH)])
            return carry

        lax.fori_loop(0, NCH, chunk_body, 0)

    return k


_sc_kernel = _make_kernel()


@jax.jit
def kernel(x, table):
    x_rows = x.reshape(B * L // GW, GW)
    return _sc_kernel(x_rows, table)


# speedup vs baseline: 8.2814x; 8.2814x over previous
import functools

import jax
import jax.numpy as jnp
from jax import lax
from jax.experimental import pallas as pl
from jax.experimental.pallas import tpu as pltpu
from jax.experimental.pallas import tpu_sc as plsc

B = 16384
L = 200
EMBED = 16
NW = 32
ROWS_PER_W = B // NW
CH = 16
IDX_PER_CH = CH * L
GW = 128
G = IDX_PER_CH // GW
NCH = ROWS_PER_W // CH


def _make_kernel():
    mesh = plsc.VectorSubcoreMesh(core_axis_name="c", subcore_axis_name="s")

    @functools.partial(
        pl.kernel,
        mesh=mesh,
        out_type=jax.ShapeDtypeStruct((B, EMBED), jnp.float32),
        compiler_params=pltpu.CompilerParams(use_tc_tiling_on_sc=False),
        scratch_types=[
            pltpu.VMEM((IDX_PER_CH,), jnp.int32),
            pltpu.VMEM((IDX_PER_CH, EMBED), jnp.float32),
            pltpu.VMEM((CH, EMBED), jnp.float32),
            pltpu.SemaphoreType.DMA,
        ],
    )
    def k(x_hbm, table_hbm, out_hbm, idx_v, buf_v, out_v, sem):
        cid = lax.axis_index("c")
        sid = lax.axis_index("s")
        wid = sid * 2 + cid

        def chunk_body(c, carry):
            i0 = (wid * NCH + c) * IDX_PER_CH
            pltpu.sync_copy(x_hbm.at[pl.ds(i0, IDX_PER_CH)], idx_v)

            def fire(j, _):
                pltpu.make_async_copy(
                    table_hbm.at[idx_v.at[pl.ds(j * GW, GW)]],
                    buf_v.at[pl.ds(j * GW, GW)],
                    sem,
                ).start()
                return _

            lax.fori_loop(0, G, fire, 0)

            def drain(j, _):
                pltpu.make_async_copy(
                    table_hbm.at[idx_v.at[pl.ds(j * GW, GW)]],
                    buf_v.at[pl.ds(j * GW, GW)],
                    sem,
                ).wait()
                return _

            lax.fori_loop(0, G, drain, 0)

            def row_body(r, _):
                base = r * L

                def acc_body(i, accs):
                    a0, a1, a2, a3 = accs
                    o = base + i * 8
                    a0 = a0 + buf_v[o + 0]
                    a1 = a1 + buf_v[o + 1]
                    a2 = a2 + buf_v[o + 2]
                    a3 = a3 + buf_v[o + 3]
                    a0 = a0 + buf_v[o + 4]
                    a1 = a1 + buf_v[o + 5]
                    a2 = a2 + buf_v[o + 6]
                    a3 = a3 + buf_v[o + 7]
                    return (a0, a1, a2, a3)

                z = jnp.zeros((16,), jnp.float32)
                a0, a1, a2, a3 = lax.fori_loop(0, L // 8, acc_body, (z, z, z, z))
                out_v[r] = ((a0 + a1) + (a2 + a3)) * (1.0 / L)
                return _

            lax.fori_loop(0, CH, row_body, 0)

            out0 = wid * ROWS_PER_W + c * CH
            pltpu.sync_copy(out_v, out_hbm.at[pl.ds(out0, CH)])
            return carry

        lax.fori_loop(0, NCH, chunk_body, 0)

    return k


_sc_kernel = _make_kernel()


@jax.jit
def kernel(x, table):
    x_flat = x.reshape(B * L)
    return _sc_kernel(x_flat, table)


# speedup vs baseline: 9.5980x; 1.1590x over previous
import functools

import jax
import jax.numpy as jnp
from jax import lax
from jax.experimental import pallas as pl
from jax.experimental.pallas import tpu as pltpu
from jax.experimental.pallas import tpu_sc as plsc

B = 16384
L = 200
EMBED = 16
NW = 32
ROWS_PER_W = B // NW
CH = 16
IDX_PER_CH = CH * L
GW = 128
G = IDX_PER_CH // GW
NCH = ROWS_PER_W // CH
NP = NCH // 2


def _make_kernel():
    mesh = plsc.VectorSubcoreMesh(core_axis_name="c", subcore_axis_name="s")

    @functools.partial(
        pl.kernel,
        mesh=mesh,
        out_type=jax.ShapeDtypeStruct((B, EMBED), jnp.float32),
        compiler_params=pltpu.CompilerParams(use_tc_tiling_on_sc=False),
        scratch_types=[
            pltpu.VMEM((IDX_PER_CH,), jnp.int32),
            pltpu.VMEM((IDX_PER_CH,), jnp.int32),
            pltpu.VMEM((IDX_PER_CH, EMBED), jnp.float32),
            pltpu.VMEM((IDX_PER_CH, EMBED), jnp.float32),
            pltpu.VMEM((CH, EMBED), jnp.float32),
            pltpu.SemaphoreType.DMA,
            pltpu.SemaphoreType.DMA,
        ],
    )
    def k(x_hbm, table_hbm, out_hbm, idx0, idx1, buf0, buf1, out_v, sem0, sem1):
        cid = lax.axis_index("c")
        sid = lax.axis_index("s")
        wid = sid * 2 + cid

        def stage_and_fire(c, idx_v, buf_v, sem):
            i0 = (wid * NCH + c) * IDX_PER_CH
            pltpu.sync_copy(x_hbm.at[pl.ds(i0, IDX_PER_CH)], idx_v)

            def fire(j, _):
                pltpu.make_async_copy(
                    table_hbm.at[idx_v.at[pl.ds(j * GW, GW)]],
                    buf_v.at[pl.ds(j * GW, GW)],
                    sem,
                ).start()
                return _

            lax.fori_loop(0, G, fire, 0)

        def drain(idx_v, buf_v, sem):
            def drain_one(j, _):
                pltpu.make_async_copy(
                    table_hbm.at[idx_v.at[pl.ds(j * GW, GW)]],
                    buf_v.at[pl.ds(j * GW, GW)],
                    sem,
                ).wait()
                return _

            lax.fori_loop(0, G, drain_one, 0)

        def consume(c, buf_v):
            def row_body(r, _):
                base = r * L

                def acc_body(i, accs):
                    a0, a1, a2, a3 = accs
                    o = base + i * 40
                    for u in range(10):
                        a0 = a0 + buf_v[o + 4 * u + 0]
                        a1 = a1 + buf_v[o + 4 * u + 1]
                        a2 = a2 + buf_v[o + 4 * u + 2]
                        a3 = a3 + buf_v[o + 4 * u + 3]
                    return (a0, a1, a2, a3)

                z = jnp.zeros((16,), jnp.float32)
                a0, a1, a2, a3 = lax.fori_loop(0, L // 40, acc_body, (z, z, z, z))
                out_v[r] = ((a0 + a1) + (a2 + a3)) * (1.0 / L)
                return _

            lax.fori_loop(0, CH, row_body, 0)

            out0 = wid * ROWS_PER_W + c * CH
            pltpu.sync_copy(out_v, out_hbm.at[pl.ds(out0, CH)])

        stage_and_fire(0, idx0, buf0, sem0)

        def pair_body(t, _):
            c0 = 2 * t
            stage_and_fire(c0 + 1, idx1, buf1, sem1)
            drain(idx0, buf0, sem0)
            consume(c0, buf0)

            @pl.when(t < NP - 1)
            def _prefetch():
                stage_and_fire(c0 + 2, idx0, buf0, sem0)

            drain(idx1, buf1, sem1)
            consume(c0 + 1, buf1)
            return _

        lax.fori_loop(0, NP, pair_body, 0)

    return k


_sc_kernel = _make_kernel()


@jax.jit
def kernel(x, table):
    x_flat = x.reshape(B * L)
    return _sc_kernel(x_flat, table)
